# baseline (speedup 1.0000x reference)
import functools

import jax
import jax.numpy as jnp
from jax import lax
from jax.experimental import pallas as pl
from jax.experimental.pallas import tpu as pltpu
from jax.experimental.pallas import tpu_sc as plsc

NC = 2
NS = 16
NB = 128
GB = 2


def _sc_mesh():
  return plsc.VectorSubcoreMesh(core_axis_name="c", subcore_axis_name="s",
                                num_cores=NC, num_subcores=NS)


def _zero_slice(zb, acc, row0, nrows):
  nfull = nrows // 64
  rem = nrows % 64

  @pl.loop(0, nfull)
  def _(i):
    pltpu.sync_copy(zb, acc.at[pl.ds(row0 + i * 64, 64)])

  if rem:
    pltpu.sync_copy(zb.at[pl.ds(0, rem)], acc.at[pl.ds(row0 + nfull * 64, rem)])


def _make_deg_kernel(half, accrows, rows_w, d):
  zrows = accrows // NS
  orows = half // NS

  @functools.partial(
      pl.kernel,
      out_type=jax.ShapeDtypeStruct((NC * half, d), jnp.float32),
      mesh=_sc_mesh(),
      scratch_types=[
          pltpu.VMEM((rows_w, NB), jnp.int32),
          pltpu.VMEM((GB * NB, d), jnp.float32),
          pltpu.VMEM((64, d), jnp.float32),
          pltpu.VMEM_SHARED((accrows, d), jnp.float32),
          pltpu.SemaphoreType.DMA,
      ],
  )
  def deg_kernel(dstc_hbm, out_hbm, didx, ones, zb, degsh, dsem):
    c = lax.axis_index("c")
    s = lax.axis_index("s")

    ov = jnp.ones((16,), jnp.float32)
    zv = jnp.zeros((16,), jnp.float32)
    nsub = d // 16

    @pl.loop(0, GB * NB * nsub)
    def _(i):
      ones[i // nsub, pl.ds((i % nsub) * 16, 16)] = ov

    @pl.loop(0, 64 * nsub)
    def _(i):
      zb[i // nsub, pl.ds((i % nsub) * 16, 16)] = zv

    _zero_slice(zb, degsh, s * zrows, zrows)
    pltpu.sync_copy(dstc_hbm.at[c, pl.ds(s * rows_w, rows_w), :], didx)
    plsc.subcore_barrier()

    @pl.loop(0, rows_w // GB)
    def _(g):
      descs = [
          pltpu.async_copy(ones.at[pl.ds(b * NB, NB)],
                           degsh.at[didx.at[g * GB + b]], dsem, add=True)
          for b in range(GB)
      ]
      for de in descs:
        de.wait()

    plsc.subcore_barrier()
    pltpu.sync_copy(degsh.at[pl.ds(s * orows, orows)],
                    out_hbm.at[pl.ds(c * half + s * orows, orows)])

  return deg_kernel


def _make_msg_kernel(half, accrows, rows_w, d):
  zrows = accrows // NS
  orows = half // NS

  @functools.partial(
      pl.kernel,
      out_type=jax.ShapeDtypeStruct((NC * half, d), jnp.float32),
      mesh=_sc_mesh(),
      scratch_types=[
          pltpu.VMEM((rows_w, NB), jnp.int32),
          pltpu.VMEM((rows_w, NB), jnp.int32),
          pltpu.VMEM((GB * NB, d), jnp.float32),
          pltpu.VMEM((64, d), jnp.float32),
          pltpu.VMEM_SHARED((accrows, d), jnp.float32),
          pltpu.SemaphoreType.DMA,
          pltpu.SemaphoreType.DMA,
      ],
  )
  def msg_kernel(src_hbm, dstc_hbm, hs_hbm, out_hbm,
                 sidx, didx, rows, zb, acc, gsem, ssem):
    c = lax.axis_index("c")
    s = lax.axis_index("s")

    zv = jnp.zeros((16,), jnp.float32)
    nsub = d // 16

    @pl.loop(0, 64 * nsub)
    def _(i):
      zb[i // nsub, pl.ds((i % nsub) * 16, 16)] = zv

    _zero_slice(zb, acc, s * zrows, zrows)
    pltpu.sync_copy(src_hbm.at[pl.ds(s * rows_w, rows_w)], sidx)
    pltpu.sync_copy(dstc_hbm.at[c, pl.ds(s * rows_w, rows_w), :], didx)
    plsc.subcore_barrier()

    @pl.loop(0, rows_w // GB)
    def _(g):
      j0 = g * GB
      gds = [
          pltpu.async_copy(hs_hbm.at[sidx.at[j0 + b]],
                           rows.at[pl.ds(b * NB, NB)], gsem)
          for b in range(GB)
      ]
      for de in gds:
        de.wait()
      sds = [
          pltpu.async_copy(rows.at[pl.ds(b * NB, NB)],
                           acc.at[didx.at[j0 + b]], ssem, add=True)
          for b in range(GB)
      ]
      for de in sds:
        de.wait()

    plsc.subcore_barrier()
    pltpu.sync_copy(acc.at[pl.ds(s * orows, orows)],
                    out_hbm.at[pl.ds(c * half + s * orows, orows)])

  return msg_kernel


def _tc_matmul(x, w):
  n, dk = x.shape
  h = w.shape[1]
  r = 1000

  def body(x_ref, w_ref, o_ref):
    o_ref[...] = jnp.dot(x_ref[...], w_ref[...],
                         preferred_element_type=jnp.float32)

  return pl.pallas_call(
      body,
      grid=(n // r,),
      in_specs=[pl.BlockSpec((r, dk), lambda i: (i, 0)),
                pl.BlockSpec((dk, h), lambda i: (0, 0))],
      out_specs=pl.BlockSpec((r, h), lambda i: (i, 0)),
      out_shape=jax.ShapeDtypeStruct((n, h), jnp.float32),
  )(x, w)


def _tc_scale(deg, h1):
  n, d = h1.shape
  r = 1000

  def body(dg_ref, h_ref, hs_ref, dis_ref):
    deg_v = dg_ref[...] + 1.0
    dis = lax.rsqrt(deg_v)
    dis_ref[...] = dis
    hs_ref[...] = h_ref[...] * dis[:, 0:1]

  return pl.pallas_call(
      body,
      grid=(n // r,),
      in_specs=[pl.BlockSpec((r, 16), lambda i: (i, 0)),
                pl.BlockSpec((r, d), lambda i: (i, 0))],
      out_specs=[pl.BlockSpec((r, d), lambda i: (i, 0)),
                 pl.BlockSpec((r, 16), lambda i: (i, 0))],
      out_shape=[jax.ShapeDtypeStruct((n, d), jnp.float32),
                 jax.ShapeDtypeStruct((n, 16), jnp.float32)],
  )(deg, h1)


def _tc_layer(acc, hs, dis, w, b):
  n, d = hs.shape
  h = w.shape[1]
  r = 1000

  def body(a_ref, hs_ref, dis_ref, w_ref, b_ref, o_ref):
    disc = dis_ref[...][:, 0:1]
    pre = disc * (a_ref[...] + hs_ref[...]) + b_ref[...]
    act = jnp.maximum(pre, 0.0)
    o_ref[...] = jnp.dot(act, w_ref[...],
                         preferred_element_type=jnp.float32) * disc

  return pl.pallas_call(
      body,
      grid=(n // r,),
      in_specs=[pl.BlockSpec((r, d), lambda i: (i, 0)),
                pl.BlockSpec((r, d), lambda i: (i, 0)),
                pl.BlockSpec((r, 16), lambda i: (i, 0)),
                pl.BlockSpec((d, h), lambda i: (0, 0)),
                pl.BlockSpec((1, d), lambda i: (0, 0))],
      out_specs=pl.BlockSpec((r, h), lambda i: (i, 0)),
      out_shape=jax.ShapeDtypeStruct((n, h), jnp.float32),
  )(acc, hs, dis, w, b)


def _tc_head(acc, hs, dis, b2, w, bo):
  n, d = hs.shape
  co = w.shape[1]
  r = 1000

  def body(a_ref, hs_ref, dis_ref, b2_ref, w_ref, bo_ref, o_ref):
    disc = dis_ref[...][:, 0:1]
    pre = disc * (a_ref[...] + hs_ref[...]) + b2_ref[...]
    act = jnp.maximum(pre, 0.0)
    z = jnp.dot(act, w_ref[...],
                preferred_element_type=jnp.float32) + bo_ref[...]
    m = jnp.max(z, axis=1, keepdims=True)
    e = jnp.exp(z - m)
    lse = jnp.log(jnp.sum(e, axis=1, keepdims=True)) + m
    o_ref[...] = z - lse

  return pl.pallas_call(
      body,
      grid=(n // r,),
      in_specs=[pl.BlockSpec((r, d), lambda i: (i, 0)),
                pl.BlockSpec((r, d), lambda i: (i, 0)),
                pl.BlockSpec((r, 16), lambda i: (i, 0)),
                pl.BlockSpec((1, d), lambda i: (0, 0)),
                pl.BlockSpec((d, co), lambda i: (0, 0)),
                pl.BlockSpec((1, co), lambda i: (0, 0))],
      out_specs=pl.BlockSpec((r, co), lambda i: (i, 0)),
      out_shape=jax.ShapeDtypeStruct((n, co), jnp.float32),
  )(acc, hs, dis, b2, w, bo)


def kernel(x, edge_index, W1, b1, W2, b2, W_out, b_out):
  n, d = x.shape
  e = edge_index.shape[1]

  half = -(-n // (2 * NS * 8)) * (NS * 8)
  accrows = half + 128
  rows_w = -(-e // (NB * NS * 8)) * 8
  rows_tot = NS * rows_w
  ep = rows_tot * NB

  src = edge_index[0]
  dst = edge_index[1]
  pad = ep - e
  srcp = jnp.concatenate([src, jnp.zeros((pad,), edge_index.dtype)])
  dstp = jnp.concatenate([dst, jnp.full((pad,), n, edge_index.dtype)])
  dst0 = jnp.minimum(dstp, half)
  dst1 = jnp.where(dstp >= half, dstp - half, half)
  src2 = srcp.reshape(rows_tot, NB)
  dstc = jnp.stack([dst0, dst1]).reshape(NC, rows_tot, NB)

  deg_kernel = _make_deg_kernel(half, accrows, rows_w, d)
  msg_kernel = _make_msg_kernel(half, accrows, rows_w, d)

  deg = lax.slice(deg_kernel(dstc), (0, 0), (n, 16))
  h1 = _tc_matmul(x, W1)
  hs1, dis = _tc_scale(deg, h1)

  acc1 = lax.slice(msg_kernel(src2, dstc, hs1), (0, 0), (n, d))
  hs2 = _tc_layer(acc1, hs1, dis, W2, b1.reshape(1, -1))

  acc2 = lax.slice(msg_kernel(src2, dstc, hs2), (0, 0), (n, d))
  out = _tc_head(acc2, hs2, dis, b2.reshape(1, -1), W_out, b_out.reshape(1, -1))
  return out


# speedup vs baseline: 1.5040x; 1.5040x over previous
import functools

import jax
import jax.numpy as jnp
from jax import lax
from jax.experimental import pallas as pl
from jax.experimental.pallas import tpu as pltpu
from jax.experimental.pallas import tpu_sc as plsc

NC = 2
NS = 16
NB = 128
GB = 1


def _sc_mesh():
  return plsc.VectorSubcoreMesh(core_axis_name="c", subcore_axis_name="s",
                                num_cores=NC, num_subcores=NS)


def _zero_slice(zb, acc, row0, nrows):
  nfull = nrows // 64
  rem = nrows % 64

  @pl.loop(0, nfull)
  def _(i):
    pltpu.sync_copy(zb, acc.at[pl.ds(row0 + i * 64, 64)])

  if rem:
    pltpu.sync_copy(zb.at[pl.ds(0, rem)], acc.at[pl.ds(row0 + nfull * 64, rem)])


def _make_deg_kernel(npad, accrows, rows_w, d):
  zrows = accrows // NS
  orows = npad // NS

  @functools.partial(
      pl.kernel,
      out_type=jax.ShapeDtypeStruct((NC * npad, d), jnp.float32),
      mesh=_sc_mesh(),
      scratch_types=[
          pltpu.VMEM((rows_w, NB), jnp.int32),
          pltpu.VMEM((GB * NB, d), jnp.float32),
          pltpu.VMEM((64, d), jnp.float32),
          pltpu.VMEM_SHARED((accrows, d), jnp.float32),
          pltpu.SemaphoreType.DMA,
      ],
  )
  def deg_kernel(dst_hbm, out_hbm, didx, ones, zb, degsh, dsem):
    c = lax.axis_index("c")
    s = lax.axis_index("s")

    ov = jnp.ones((16,), jnp.float32)
    zv = jnp.zeros((16,), jnp.float32)
    nsub = d // 16

    @pl.loop(0, GB * NB * nsub)
    def _(i):
      ones[i // nsub, pl.ds((i % nsub) * 16, 16)] = ov

    @pl.loop(0, 64 * nsub)
    def _(i):
      zb[i // nsub, pl.ds((i % nsub) * 16, 16)] = zv

    _zero_slice(zb, degsh, s * zrows, zrows)
    pltpu.sync_copy(dst_hbm.at[c, pl.ds(s * rows_w, rows_w), :], didx)
    plsc.subcore_barrier()

    @pl.loop(0, rows_w // GB)
    def _(g):
      descs = [
          pltpu.async_copy(ones.at[pl.ds(b * NB, NB)],
                           degsh.at[didx.at[g * GB + b]], dsem, add=True)
          for b in range(GB)
      ]
      for de in descs:
        de.wait()

    plsc.subcore_barrier()
    pltpu.sync_copy(degsh.at[pl.ds(s * orows, orows)],
                    out_hbm.at[pl.ds(c * npad + s * orows, orows)])

  return deg_kernel


def _make_msg_kernel(npad, accrows, rows_w, d):
  zrows = accrows // NS
  orows = npad // NS

  @functools.partial(
      pl.kernel,
      out_type=jax.ShapeDtypeStruct((NC * npad, d), jnp.float32),
      mesh=_sc_mesh(),
      scratch_types=[
          pltpu.VMEM((rows_w, NB), jnp.int32),
          pltpu.VMEM((rows_w, NB), jnp.int32),
          pltpu.VMEM((GB * NB, d), jnp.float32),
          pltpu.VMEM((64, d), jnp.float32),
          pltpu.VMEM_SHARED((accrows, d), jnp.float32),
          pltpu.SemaphoreType.DMA,
          pltpu.SemaphoreType.DMA,
      ],
  )
  def msg_kernel(src_hbm, dst_hbm, hs_hbm, out_hbm,
                 sidx, didx, rows, zb, acc, gsem, ssem):
    c = lax.axis_index("c")
    s = lax.axis_index("s")

    zv = jnp.zeros((16,), jnp.float32)
    nsub = d // 16

    @pl.loop(0, 64 * nsub)
    def _(i):
      zb[i // nsub, pl.ds((i % nsub) * 16, 16)] = zv

    _zero_slice(zb, acc, s * zrows, zrows)
    pltpu.sync_copy(src_hbm.at[c, pl.ds(s * rows_w, rows_w), :], sidx)
    pltpu.sync_copy(dst_hbm.at[c, pl.ds(s * rows_w, rows_w), :], didx)
    plsc.subcore_barrier()

    @pl.loop(0, rows_w // GB)
    def _(g):
      j0 = g * GB
      gds = [
          pltpu.async_copy(hs_hbm.at[sidx.at[j0 + b]],
                           rows.at[pl.ds(b * NB, NB)], gsem)
          for b in range(GB)
      ]
      for de in gds:
        de.wait()
      sds = [
          pltpu.async_copy(rows.at[pl.ds(b * NB, NB)],
                           acc.at[didx.at[j0 + b]], ssem, add=True)
          for b in range(GB)
      ]
      for de in sds:
        de.wait()

    plsc.subcore_barrier()
    pltpu.sync_copy(acc.at[pl.ds(s * orows, orows)],
                    out_hbm.at[pl.ds(c * npad + s * orows, orows)])

  return msg_kernel


def _tc_matmul(x, w):
  n, dk = x.shape
  h = w.shape[1]
  r = 1000

  def body(x_ref, w_ref, o_ref):
    o_ref[...] = jnp.dot(x_ref[...], w_ref[...],
                         preferred_element_type=jnp.float32)

  return pl.pallas_call(
      body,
      grid=(n // r,),
      in_specs=[pl.BlockSpec((r, dk), lambda i: (i, 0)),
                pl.BlockSpec((dk, h), lambda i: (0, 0))],
      out_specs=pl.BlockSpec((r, h), lambda i: (i, 0)),
      out_shape=jax.ShapeDtypeStruct((n, h), jnp.float32),
  )(x, w)


def _tc_scale(deg0, deg1, h1):
  n, d = h1.shape
  r = 1000

  def body(d0_ref, d1_ref, h_ref, hs_ref, dis_ref):
    deg_v = d0_ref[...] + d1_ref[...] + 1.0
    dis = lax.rsqrt(deg_v)
    dis_ref[...] = dis
    hs_ref[...] = h_ref[...] * dis[:, 0:1]

  return pl.pallas_call(
      body,
      grid=(n // r,),
      in_specs=[pl.BlockSpec((r, 16), lambda i: (i, 0)),
                pl.BlockSpec((r, 16), lambda i: (i, 0)),
                pl.BlockSpec((r, d), lambda i: (i, 0))],
      out_specs=[pl.BlockSpec((r, d), lambda i: (i, 0)),
                 pl.BlockSpec((r, 16), lambda i: (i, 0))],
      out_shape=[jax.ShapeDtypeStruct((n, d), jnp.float32),
                 jax.ShapeDtypeStruct((n, 16), jnp.float32)],
  )(deg0, deg1, h1)


def _tc_layer(acc0, acc1, hs, dis, w, b):
  n, d = hs.shape
  h = w.shape[1]
  r = 1000

  def body(a0_ref, a1_ref, hs_ref, dis_ref, w_ref, b_ref, o_ref):
    disc = dis_ref[...][:, 0:1]
    pre = disc * (a0_ref[...] + a1_ref[...] + hs_ref[...]) + b_ref[...]
    act = jnp.maximum(pre, 0.0)
    o_ref[...] = jnp.dot(act, w_ref[...],
                         preferred_element_type=jnp.float32) * disc

  return pl.pallas_call(
      body,
      grid=(n // r,),
      in_specs=[pl.BlockSpec((r, d), lambda i: (i, 0)),
                pl.BlockSpec((r, d), lambda i: (i, 0)),
                pl.BlockSpec((r, d), lambda i: (i, 0)),
                pl.BlockSpec((r, 16), lambda i: (i, 0)),
                pl.BlockSpec((d, h), lambda i: (0, 0)),
                pl.BlockSpec((1, d), lambda i: (0, 0))],
      out_specs=pl.BlockSpec((r, h), lambda i: (i, 0)),
      out_shape=jax.ShapeDtypeStruct((n, h), jnp.float32),
  )(acc0, acc1, hs, dis, w, b)


def _tc_head(acc0, acc1, hs, dis, b2, w, bo):
  n, d = hs.shape
  co = w.shape[1]
  r = 1000

  def body(a0_ref, a1_ref, hs_ref, dis_ref, b2_ref, w_ref, bo_ref, o_ref):
    disc = dis_ref[...][:, 0:1]
    pre = disc * (a0_ref[...] + a1_ref[...] + hs_ref[...]) + b2_ref[...]
    act = jnp.maximum(pre, 0.0)
    z = jnp.dot(act, w_ref[...],
                preferred_element_type=jnp.float32) + bo_ref[...]
    m = jnp.max(z, axis=1, keepdims=True)
    e = jnp.exp(z - m)
    lse = jnp.log(jnp.sum(e, axis=1, keepdims=True)) + m
    o_ref[...] = z - lse

  return pl.pallas_call(
      body,
      grid=(n // r,),
      in_specs=[pl.BlockSpec((r, d), lambda i: (i, 0)),
                pl.BlockSpec((r, d), lambda i: (i, 0)),
                pl.BlockSpec((r, d), lambda i: (i, 0)),
                pl.BlockSpec((r, 16), lambda i: (i, 0)),
                pl.BlockSpec((1, d), lambda i: (0, 0)),
                pl.BlockSpec((d, co), lambda i: (0, 0)),
                pl.BlockSpec((1, co), lambda i: (0, 0))],
      out_specs=pl.BlockSpec((r, co), lambda i: (i, 0)),
      out_shape=jax.ShapeDtypeStruct((n, co), jnp.float32),
  )(acc0, acc1, hs, dis, b2, w, bo)


def kernel(x, edge_index, W1, b1, W2, b2, W_out, b_out):
  n, d = x.shape
  e = edge_index.shape[1]

  npad = -(-n // (NS * 8)) * (NS * 8)
  accrows = npad + 128
  rows_w = -(-e // (NC * NB * NS * 8)) * 8
  rows_tot = NS * rows_w
  ep = NC * rows_tot * NB

  src = edge_index[0]
  dst = edge_index[1]
  pad = ep - e
  pidx = jnp.arange(pad, dtype=edge_index.dtype)
  srcp = jnp.concatenate([src, jnp.zeros((pad,), edge_index.dtype)])
  dstp = jnp.concatenate([dst, npad + (pidx & 127)])
  src3 = srcp.reshape(NC, rows_tot, NB)
  dst3 = dstp.reshape(NC, rows_tot, NB)

  deg_kernel = _make_deg_kernel(npad, accrows, rows_w, d)
  msg_kernel = _make_msg_kernel(npad, accrows, rows_w, d)

  def _split(arr, w):
    return (lax.slice(arr, (0, 0), (n, w)),
            lax.slice(arr, (npad, 0), (npad + n, w)))

  deg0, deg1 = _split(deg_kernel(dst3), 16)
  h1 = _tc_matmul(x, W1)
  hs1, dis = _tc_scale(deg0, deg1, h1)

  a10, a11 = _split(msg_kernel(src3, dst3, hs1), d)
  hs2 = _tc_layer(a10, a11, hs1, dis, W2, b1.reshape(1, -1))

  a20, a21 = _split(msg_kernel(src3, dst3, hs2), d)
  out = _tc_head(a20, a21, hs2, dis, b2.reshape(1, -1), W_out,
                 b_out.reshape(1, -1))
  return out
